# baseline (speedup 1.0000x reference)
import functools

import jax
import jax.numpy as jnp
from jax import lax
from jax.experimental import pallas as pl
from jax.experimental.pallas import tpu as pltpu
from jax.experimental.pallas import tpu_sc as plsc

VOCAB_SIZE = 100000
DIM = 1024
MAX_SEQ_LEN = 8192
BATCH = 4
SEQ_LEN = 4096

NC = 2
NS = 16
LANES = 16
NW = NC * NS
TOKENS = BATCH * SEQ_LEN
PER_W = TOKENS // NW
CHUNK = 32
NCHUNK = PER_W // CHUNK
VECS_PER_ROW = DIM // LANES


def _emb_kernel(ids_hbm, table_hbm, pos_hbm, out_hbm,
                idx_v, tok_v, pos_v, sem_g, sem_p):
    wid = lax.axis_index("s") * NC + lax.axis_index("c")
    base = wid * PER_W
    pos_base = lax.rem(base, SEQ_LEN)

    pltpu.sync_copy(ids_hbm.at[wid], idx_v)

    def chunk_body(c, carry):
        g = pltpu.async_copy(table_hbm.at[idx_v.at[c]], tok_v, sem_g)
        p = pltpu.async_copy(pos_hbm.at[pl.ds(pos_base + c * CHUNK, CHUNK)],
                             pos_v, sem_p)
        g.wait()
        p.wait()

        def row_body(r, rc):
            for j in range(VECS_PER_ROW):
                s = pl.ds(j * LANES, LANES)
                tok_v[r, s] = tok_v[r, s] + pos_v[r, s]
            return rc

        lax.fori_loop(0, CHUNK, row_body, 0)
        pltpu.sync_copy(tok_v, out_hbm.at[pl.ds(base + c * CHUNK, CHUNK)])
        return carry

    lax.fori_loop(0, NCHUNK, chunk_body, 0)


@jax.jit
def kernel(input_ids, token_embed_weight, pos_embed_weight):
    ids = input_ids.reshape(NW, NCHUNK, CHUNK).astype(jnp.int32)
    mesh = plsc.VectorSubcoreMesh(core_axis_name="c", subcore_axis_name="s")
    out = pl.kernel(
        _emb_kernel,
        out_type=jax.ShapeDtypeStruct((TOKENS, DIM), jnp.float32),
        mesh=mesh,
        scratch_types=[
            pltpu.VMEM((NCHUNK, CHUNK), jnp.int32),
            pltpu.VMEM((CHUNK, DIM), jnp.float32),
            pltpu.VMEM((CHUNK, DIM), jnp.float32),
            pltpu.SemaphoreType.DMA,
            pltpu.SemaphoreType.DMA,
        ],
    )(ids, token_embed_weight, pos_embed_weight)
    return out.reshape(BATCH, SEQ_LEN, DIM)
